# Initial kernel scaffold; baseline (speedup 1.0000x reference)
import jax
import jax.numpy as jnp
from jax.experimental import pallas as pl


def kernel(x, emb_tables, lin_tables, bias):
    raise NotImplementedError("write your pallas kernel here")


# speedup vs baseline: 1.5984x; 1.5984x over previous
import functools

import jax
import jax.numpy as jnp
from jax import lax
from jax.experimental import pallas as pl
from jax.experimental.pallas import tpu as pltpu
from jax.experimental.pallas import tpu_sc as plsc

F = 26
V = 100000
D = 32
B = 16384

NW = 32
PER_W = B // NW
SW = 104
CB = 32
ROWS_C = CB * F
SPC = ROWS_C // SW
NCH = PER_W // CB
ROWS_W = PER_W * F // SW

_mesh = plsc.VectorSubcoreMesh(core_axis_name="c", subcore_axis_name="s")


@functools.partial(
    pl.kernel,
    out_type=jax.ShapeDtypeStruct((B,), jnp.float32),
    mesh=_mesh,
    compiler_params=pltpu.CompilerParams(use_tc_tiling_on_sc=False),
    scratch_types=[
        pltpu.VMEM((ROWS_W, SW), jnp.int32),
        pltpu.VMEM((ROWS_C, D), jnp.float32),
        pltpu.VMEM((ROWS_C, D), jnp.float32),
        pltpu.VMEM((ROWS_C + 16,), jnp.float32),
        pltpu.VMEM((ROWS_C + 16,), jnp.float32),
        pltpu.VMEM((PER_W,), jnp.float32),
        pltpu.VMEM((16,), jnp.float32),
        pltpu.SemaphoreType.DMA,
        pltpu.SemaphoreType.DMA,
    ],
)
def _fm_sc(idx_hbm, emb_hbm, lin_hbm, bias_hbm, out_hbm,
           idxv, rows0, rows1, lin0b, lin1b, outv, biasv, sem_e, sem_l):
    wid = lax.axis_index("s") * 2 + lax.axis_index("c")
    pltpu.sync_copy(idx_hbm.at[pl.ds(wid * ROWS_W, ROWS_W)], idxv)
    pltpu.sync_copy(bias_hbm, biasv)

    rowsb = [rows0, rows1]
    linb = [lin0b, lin1b]
    lane = lax.iota(jnp.int32, 16)
    tail_mask = lane < (F - 16)
    _gd = lax.GatherDimensionNumbers(
        offset_dims=(), collapsed_slice_dims=(0,), start_index_map=(0,))

    def shuffle(v, perm):
        return lax.gather(v, perm[:, None], _gd, slice_sizes=(1,),
                          mode=lax.GatherScatterMode.PROMISE_IN_BOUNDS)

    def fire(c, p):
        cps = []
        for j in range(SPC):
            r = c * SPC + j
            cps.append(pltpu.async_copy(
                emb_hbm.at[idxv.at[r]],
                rowsb[p].at[pl.ds(j * SW, SW), :], sem_e))
            cps.append(pltpu.async_copy(
                lin_hbm.at[idxv.at[r]],
                linb[p].at[pl.ds(j * SW, SW)], sem_l))
        return cps

    def compute(c, p):
        rowsv = rowsb[p]
        linv = linb[p]

        def group(g, carry):
            def body(jj, zacc):
                base = (g * 16 + jj) * F
                s0 = jnp.zeros((16,), jnp.float32)
                s1 = jnp.zeros((16,), jnp.float32)
                q0 = jnp.zeros((16,), jnp.float32)
                q1 = jnp.zeros((16,), jnp.float32)
                for i in range(F):
                    r0 = rowsv[base + i, pl.ds(0, 16)]
                    r1 = rowsv[base + i, pl.ds(16, 16)]
                    s0 = s0 + r0
                    s1 = s1 + r1
                    q0 = q0 + r0 * r0
                    q1 = q1 + r1 * r1
                lin0 = linv[pl.ds(base, 16)]
                lin1 = linv[pl.ds(base + 16, 16)]
                vec = 0.5 * (s0 * s0 + s1 * s1 - q0 - q1)
                vec = vec + lin0 + jnp.where(tail_mask, lin1, 0.0)
                for k in (8, 4, 2, 1):
                    vec = vec + shuffle(vec, lane ^ k)
                return jnp.where(lane == jj, vec, zacc)
            zacc = lax.fori_loop(0, 16, body, jnp.zeros((16,), jnp.float32))
            outv[pl.ds(c * CB + g * 16, 16)] = zacc
            return carry
        lax.fori_loop(0, CB // 16, group, 0)

    for c in range(NCH):
        cps = fire(c, c % 2)
        if c > 0:
            for cp in prev:
                cp.wait()
            compute(c - 1, (c - 1) % 2)
        prev = cps
    for cp in prev:
        cp.wait()
    compute(NCH - 1, (NCH - 1) % 2)

    bvec = biasv[...]
    for k in range(PER_W // 16):
        z = outv[pl.ds(k * 16, 16)] + bvec
        outv[pl.ds(k * 16, 16)] = 1.0 / (1.0 + jnp.exp(-z))
    pltpu.sync_copy(outv, out_hbm.at[pl.ds(wid * PER_W, PER_W)])


def kernel(x, emb_tables, lin_tables, bias):
    off = (jnp.arange(F, dtype=jnp.int32) * V)[None, :]
    idx = (x + off).reshape(B * F // SW, SW)
    emb_flat = emb_tables.reshape(F * V, D)
    lin_flat = lin_tables.reshape(F * V)
    bias16 = jnp.broadcast_to(bias, (16,))
    out = _fm_sc(idx, emb_flat, lin_flat, bias16)
    return out.reshape(-1, 1)
